# baseline (speedup 1.0000x reference)
import functools

import jax
import jax.numpy as jnp
from jax import lax
from jax.experimental import pallas as pl
from jax.experimental.pallas import tpu as pltpu
from jax.experimental.pallas import tpu_sc as plsc

N = 10000
E = 320000
R = 16
NB = 4
H = 128
HID = 256

NC = 2
NS = 16
NT = NC * NS
NPT = 320
NPAD = NT * NPT
LK = NPT * R

CH = 4000
GCH = 128
ECAP = 128 * 94

_BLKN = 1000


def _xw_body(comp_ref, basis_ref, x_ref, out_ref):
    r = pl.program_id(0)
    w = comp_ref[r, 0] * basis_ref[0]
    for b in range(1, NB):
        w = w + comp_ref[r, b] * basis_ref[b]
    out_ref[:] = jnp.dot(x_ref[:], w, preferred_element_type=jnp.float32)


def _make_xw(comp, basis, x):
    nbn = N // _BLKN
    return pl.pallas_call(
        _xw_body,
        grid=(R, nbn),
        in_specs=[
            pl.BlockSpec(memory_space=pltpu.SMEM),
            pl.BlockSpec((NB, H, H), lambda r, n: (0, 0, 0)),
            pl.BlockSpec((_BLKN, H), lambda r, n: (n, 0)),
        ],
        out_specs=pl.BlockSpec((_BLKN, H), lambda r, n: (r * nbn + n, 0)),
        out_shape=jax.ShapeDtypeStruct((R * N, H), jnp.float32),
    )(comp, basis, x)


def _keys_body(ei_ref, et_ref, pk_ref):
    src = ei_ref[0:1, :]
    dst = ei_ref[1:2, :]
    et = et_ref[:]
    pk_ref[:] = (src << 18) | (dst * R + et)


def _make_keys(edge_index, edge_type):
    be = 12800
    pk = pl.pallas_call(
        _keys_body,
        grid=(E // be,),
        in_specs=[
            pl.BlockSpec((2, be), lambda i: (0, i)),
            pl.BlockSpec((1, be), lambda i: (0, i)),
        ],
        out_specs=pl.BlockSpec((1, be), lambda i: (0, i)),
        out_shape=jax.ShapeDtypeStruct((1, E), jnp.int32),
    )(edge_index, edge_type.reshape(1, E))
    return pk.reshape(E)


def _sc_agg_body(pk_hbm, xw_hbm, out_hbm,
                 cnt_v, elk_v, egx_v, kbuf0, kbuf1,
                 rows0, rows1, agg_v, ksem0, ksem1, rsem0, rsem1):
    wid = lax.axis_index("s") * NC + lax.axis_index("c")
    kbase = wid * LK
    base_node = wid * NPT
    kbufs = (kbuf0, kbuf1)
    rows = (rows0, rows1)
    ksems = (ksem0, ksem1)
    rsems = (rsem0, rsem1)

    zf = jnp.zeros((16,), jnp.float32)
    zi = jnp.zeros((16,), jnp.int32)
    lkfill = jnp.full((16,), LK, jnp.int32)

    def zero_cnt(i, _):
        cnt_v[pl.ds(i * 16, 16)] = zf
        return 0
    lax.fori_loop(0, (LK + 128) // 16, zero_cnt, 0)

    def zero_agg(i, _):
        agg_v[i // 8, pl.ds((i % 8) * 16, 16)] = zf
        return 0
    lax.fori_loop(0, (NPT + 1) * 8, zero_agg, 0)

    def init_lists(i, _):
        egx_v[pl.ds(i * 16, 16)] = zi
        elk_v[pl.ds(i * 16, 16)] = lkfill
        return 0
    lax.fori_loop(0, ECAP // 16, init_lists, 0)

    NCH1 = E // CH

    def p1_issue(c, par):
        pltpu.async_copy(pk_hbm.at[pl.ds(c * CH, CH)], kbufs[par],
                         ksems[par])

    p1_issue(0, 0)
    p1_issue(1, 1)

    def scan_pair(c2, nv):
        for par in range(2):
            c = 2 * c2 + par
            pltpu.make_async_copy(pk_hbm.at[pl.ds(c * CH, CH)],
                                  kbufs[par], ksems[par]).wait()
            kb = kbufs[par]

            @plsc.parallel_loop(0, CH // 16, 1, unroll=8, carry=nv)
            def nv(j, nv):
                p = kb[pl.ds(j * 16, 16)]
                k = p & 0x3FFFF
                g = (k & (R - 1)) * N + ((p >> 18) & 0x3FFF)
                lk = k - kbase
                m = (lk >= 0) & (lk < LK)
                plsc.store_compressed(elk_v.at[pl.ds(nv, 16)], lk, mask=m)
                plsc.store_compressed(egx_v.at[pl.ds(nv, 16)], g, mask=m)
                nm = plsc.all_reduce_population_count(m)
                return jnp.minimum(nv + nm[0], ECAP - 16)

            @pl.when(c + 2 < NCH1)
            def _():
                p1_issue(c + 2, par)
        return nv

    nv = lax.fori_loop(0, NCH1 // 2, scan_pair, jnp.int32(0))

    ng = (nv + 15) >> 4

    def cntloop(i, _):
        lkv = elk_v[pl.ds(i * 16, 16)]
        dupv, lastm = plsc.scan_count(lkv)
        plsc.addupdate_scatter(cnt_v, [lkv], dupv.astype(jnp.float32),
                               mask=lastm)
        return 0
    lax.fori_loop(0, ng, cntloop, 0)

    def invert(i, _):
        v = cnt_v[pl.ds(i * 16, 16)]
        cnt_v[pl.ds(i * 16, 16)] = 1.0 / jnp.maximum(v, 1.0)
        return 0
    lax.fori_loop(0, (LK + 128) // 16, invert, 0)

    nch = (nv + GCH - 1) // GCH

    def p2_issue(t, par):
        pltpu.async_copy(xw_hbm.at[egx_v.at[pl.ds(t * GCH, GCH)]],
                         rows[par], rsems[par])

    @pl.when(nch > 0)
    def _():
        p2_issue(0, 0)

    @pl.when(nch > 1)
    def _():
        p2_issue(1, 1)

    def proc_pair(t2, _):
        for par in range(2):
            t = 2 * t2 + par

            @pl.when(t < nch)
            def _():
                pltpu.make_async_copy(
                    xw_hbm.at[egx_v.at[pl.ds(t * GCH, GCH)]],
                    rows[par], rsems[par]).wait()
                off = t * GCH
                rv = rows[par]

                @plsc.parallel_loop(0, GCH // 16, 1, unroll=4)
                def grp(g):
                    lkv = elk_v[pl.ds(off + g * 16, 16)]
                    wv = plsc.load_gather(cnt_v, [lkv])
                    nodev = lkv >> 4
                    rbase = g * 16
                    for j in range(16):
                        node = nodev[j]
                        w = wv[j]
                        for h8 in range(8):
                            sl = pl.ds(h8 * 16, 16)
                            plsc.addupdate(agg_v.at[node, sl],
                                           w * rv[rbase + j, sl])

                @pl.when(t + 2 < nch)
                def _():
                    p2_issue(t + 2, par)
        return 0

    lax.fori_loop(0, (nch + 1) // 2, proc_pair, 0)

    pltpu.sync_copy(agg_v.at[pl.ds(0, NPT)], out_hbm.at[pl.ds(base_node, NPT)])


@functools.partial(
    pl.kernel,
    out_type=jax.ShapeDtypeStruct((NPAD, H), jnp.float32),
    mesh=plsc.VectorSubcoreMesh(core_axis_name="c", subcore_axis_name="s"),
    compiler_params=pltpu.CompilerParams(needs_layout_passes=False),
    scratch_types=[
        pltpu.VMEM((LK + 128,), jnp.float32),
        pltpu.VMEM((ECAP,), jnp.int32),
        pltpu.VMEM((ECAP,), jnp.int32),
        pltpu.VMEM((CH,), jnp.int32),
        pltpu.VMEM((CH,), jnp.int32),
        pltpu.VMEM((GCH, H), jnp.float32),
        pltpu.VMEM((GCH, H), jnp.float32),
        pltpu.VMEM((NPT + 1, H), jnp.float32),
        pltpu.SemaphoreType.DMA,
        pltpu.SemaphoreType.DMA,
        pltpu.SemaphoreType.DMA,
        pltpu.SemaphoreType.DMA,
    ],
)
def _sc_agg(pk_hbm, xw_hbm, out_hbm,
            cnt_v, elk_v, egx_v, kbuf0, kbuf1,
            rows0, rows1, agg_v, ksem0, ksem1, rsem0, rsem1):
    _sc_agg_body(pk_hbm, xw_hbm, out_hbm,
                 cnt_v, elk_v, egx_v, kbuf0, kbuf1,
                 rows0, rows1, agg_v, ksem0, ksem1, rsem0, rsem1)


def _epilogue_body(agg_ref, x_ref, root_ref, bias_ref, w1_ref, b1_ref,
                   w2_ref, b2_ref, w3_ref, b3_ref, out_ref):
    x = x_ref[:]
    ent = agg_ref[:] + jnp.dot(x, root_ref[:], preferred_element_type=jnp.float32)
    ent = ent + bias_ref[:] + x
    h = jnp.maximum(jnp.dot(ent, w1_ref[:], preferred_element_type=jnp.float32) + b1_ref[:], 0.0)
    p = jnp.dot(h, w2_ref[:], preferred_element_type=jnp.float32) + b2_ref[:]
    ent = p + ent
    out_ref[:] = jnp.dot(ent, w3_ref[:], preferred_element_type=jnp.float32) + b3_ref[:]


def _epilogue(agg, x, root, bias, W1, b1, W2, b2, W3, b3):
    full = lambda shape: pl.BlockSpec(shape, lambda i: (0,) * len(shape))
    return pl.pallas_call(
        _epilogue_body,
        grid=(N // _BLKN,),
        in_specs=[
            pl.BlockSpec((_BLKN, H), lambda i: (i, 0)),
            pl.BlockSpec((_BLKN, H), lambda i: (i, 0)),
            full((H, H)),
            full((1, H)),
            full((H, H // 2)),
            full((1, H // 2)),
            full((H // 2, H)),
            full((1, H)),
            full((H, HID)),
            full((1, HID)),
        ],
        out_specs=pl.BlockSpec((_BLKN, HID), lambda i: (i, 0)),
        out_shape=jax.ShapeDtypeStruct((N, HID), jnp.float32),
    )(agg, x, root, bias.reshape(1, H), W1, b1.reshape(1, H // 2),
      W2, b2.reshape(1, H), W3, b3.reshape(1, HID))


def kernel(node_embeds, basis, comp, root, rgcn_bias, W1, b1, W2, b2, W3, b3,
           edge_index, edge_type):
    x = node_embeds
    xw = _make_xw(comp, basis, x)
    pk = _make_keys(edge_index, edge_type)
    agg = _sc_agg(pk, xw)
    return _epilogue(agg, x, root, rgcn_bias, W1, b1, W2, b2, W3, b3)


# speedup vs baseline: 1.0047x; 1.0047x over previous
import functools

import jax
import jax.numpy as jnp
from jax import lax
from jax.experimental import pallas as pl
from jax.experimental.pallas import tpu as pltpu
from jax.experimental.pallas import tpu_sc as plsc

N = 10000
E = 320000
R = 16
NB = 4
H = 128
HID = 256

NC = 2
NS = 16
NT = NC * NS
NPT = 320
NPAD = NT * NPT
LK = NPT * R

CH = 4000
GCH = 128
ECAP = 128 * 94

_BLKN = 1000


def _xw_body(comp_ref, basis_ref, x_ref, out_ref):
    r = pl.program_id(0)
    w = comp_ref[r, 0] * basis_ref[0]
    for b in range(1, NB):
        w = w + comp_ref[r, b] * basis_ref[b]
    out_ref[:] = jnp.dot(x_ref[:], w, preferred_element_type=jnp.float32)


def _make_xw(comp, basis, x):
    nbn = N // _BLKN
    return pl.pallas_call(
        _xw_body,
        grid=(R, nbn),
        in_specs=[
            pl.BlockSpec(memory_space=pltpu.SMEM),
            pl.BlockSpec((NB, H, H), lambda r, n: (0, 0, 0)),
            pl.BlockSpec((_BLKN, H), lambda r, n: (n, 0)),
        ],
        out_specs=pl.BlockSpec((_BLKN, H), lambda r, n: (r * nbn + n, 0)),
        out_shape=jax.ShapeDtypeStruct((R * N, H), jnp.float32),
    )(comp, basis, x)


def _keys_body(ei_ref, et_ref, pk_ref):
    src = ei_ref[0:1, :]
    dst = ei_ref[1:2, :]
    et = et_ref[:]
    pk_ref[:] = (src << 18) | (dst * R + et)


def _make_keys(edge_index, edge_type):
    be = 12800
    pk = pl.pallas_call(
        _keys_body,
        grid=(E // be,),
        in_specs=[
            pl.BlockSpec((2, be), lambda i: (0, i)),
            pl.BlockSpec((1, be), lambda i: (0, i)),
        ],
        out_specs=pl.BlockSpec((1, be), lambda i: (0, i)),
        out_shape=jax.ShapeDtypeStruct((1, E), jnp.int32),
    )(edge_index, edge_type.reshape(1, E))
    return pk.reshape(E)


def _sc_agg_body(pk_hbm, xw_hbm, out_hbm,
                 cnt_v, elk_v, egx_v, kbuf0, kbuf1,
                 rows0, rows1, agg_v, ksem0, ksem1, rsem0, rsem1):
    wid = lax.axis_index("s") * NC + lax.axis_index("c")
    kbase = wid * LK
    base_node = wid * NPT
    kbufs = (kbuf0, kbuf1)
    rows = (rows0, rows1)
    ksems = (ksem0, ksem1)
    rsems = (rsem0, rsem1)

    zf = jnp.zeros((16,), jnp.float32)
    zi = jnp.zeros((16,), jnp.int32)
    lkfill = jnp.full((16,), LK, jnp.int32)

    def zero_cnt(i, _):
        cnt_v[pl.ds(i * 16, 16)] = zf
        return 0
    lax.fori_loop(0, (LK + 128) // 16, zero_cnt, 0)

    def zero_agg(i, _):
        agg_v[i // 8, pl.ds((i % 8) * 16, 16)] = zf
        return 0
    lax.fori_loop(0, (NPT + 1) * 8, zero_agg, 0)

    def init_lists(i, _):
        egx_v[pl.ds(i * 16, 16)] = zi
        elk_v[pl.ds(i * 16, 16)] = lkfill
        return 0
    lax.fori_loop(0, ECAP // 16, init_lists, 0)

    NCH1 = E // CH

    def p1_issue(c, par):
        pltpu.async_copy(pk_hbm.at[pl.ds(c * CH, CH)], kbufs[par],
                         ksems[par])

    p1_issue(0, 0)
    p1_issue(1, 1)

    def scan_pair(c2, nv):
        for par in range(2):
            c = 2 * c2 + par
            pltpu.make_async_copy(pk_hbm.at[pl.ds(c * CH, CH)],
                                  kbufs[par], ksems[par]).wait()
            kb = kbufs[par]

            @plsc.parallel_loop(0, CH // 16, 1, unroll=4, carry=nv)
            def nv(j, nv):
                p = kb[pl.ds(j * 16, 16)]
                k = p & 0x3FFFF
                g = (k & (R - 1)) * N + ((p >> 18) & 0x3FFF)
                lk = k - kbase
                m = (lk >= 0) & (lk < LK)
                plsc.store_compressed(elk_v.at[pl.ds(nv, 16)], lk, mask=m)
                plsc.store_compressed(egx_v.at[pl.ds(nv, 16)], g, mask=m)
                nm = plsc.all_reduce_population_count(m)
                return jnp.minimum(nv + nm[0], ECAP - 16)

            @pl.when(c + 2 < NCH1)
            def _():
                p1_issue(c + 2, par)
        return nv

    nv = lax.fori_loop(0, NCH1 // 2, scan_pair, jnp.int32(0))

    ng = (nv + 15) >> 4

    def cntloop(i, _):
        lkv = elk_v[pl.ds(i * 16, 16)]
        dupv, lastm = plsc.scan_count(lkv)
        plsc.addupdate_scatter(cnt_v, [lkv], dupv.astype(jnp.float32),
                               mask=lastm)
        return 0
    lax.fori_loop(0, ng, cntloop, 0)

    def invert(i, _):
        v = cnt_v[pl.ds(i * 16, 16)]
        cnt_v[pl.ds(i * 16, 16)] = 1.0 / jnp.maximum(v, 1.0)
        return 0
    lax.fori_loop(0, (LK + 128) // 16, invert, 0)

    nch = (nv + GCH - 1) // GCH

    def p2_issue(t, par):
        pltpu.async_copy(xw_hbm.at[egx_v.at[pl.ds(t * GCH, GCH)]],
                         rows[par], rsems[par])

    @pl.when(nch > 0)
    def _():
        p2_issue(0, 0)

    @pl.when(nch > 1)
    def _():
        p2_issue(1, 1)

    def proc_pair(t2, _):
        for par in range(2):
            t = 2 * t2 + par

            @pl.when(t < nch)
            def _():
                pltpu.make_async_copy(
                    xw_hbm.at[egx_v.at[pl.ds(t * GCH, GCH)]],
                    rows[par], rsems[par]).wait()
                off = t * GCH
                rv = rows[par]

                @plsc.parallel_loop(0, GCH // 16, 1, unroll=4)
                def grp(g):
                    lkv = elk_v[pl.ds(off + g * 16, 16)]
                    wv = plsc.load_gather(cnt_v, [lkv])
                    nodev = lkv >> 4
                    rbase = g * 16
                    for j in range(16):
                        node = nodev[j]
                        w = wv[j]
                        for h8 in range(8):
                            sl = pl.ds(h8 * 16, 16)
                            plsc.addupdate(agg_v.at[node, sl],
                                           w * rv[rbase + j, sl])

                @pl.when(t + 2 < nch)
                def _():
                    p2_issue(t + 2, par)
        return 0

    lax.fori_loop(0, (nch + 1) // 2, proc_pair, 0)

    pltpu.sync_copy(agg_v.at[pl.ds(0, NPT)], out_hbm.at[pl.ds(base_node, NPT)])


@functools.partial(
    pl.kernel,
    out_type=jax.ShapeDtypeStruct((NPAD, H), jnp.float32),
    mesh=plsc.VectorSubcoreMesh(core_axis_name="c", subcore_axis_name="s"),
    compiler_params=pltpu.CompilerParams(needs_layout_passes=False),
    scratch_types=[
        pltpu.VMEM((LK + 128,), jnp.float32),
        pltpu.VMEM((ECAP,), jnp.int32),
        pltpu.VMEM((ECAP,), jnp.int32),
        pltpu.VMEM((CH,), jnp.int32),
        pltpu.VMEM((CH,), jnp.int32),
        pltpu.VMEM((GCH, H), jnp.float32),
        pltpu.VMEM((GCH, H), jnp.float32),
        pltpu.VMEM((NPT + 1, H), jnp.float32),
        pltpu.SemaphoreType.DMA,
        pltpu.SemaphoreType.DMA,
        pltpu.SemaphoreType.DMA,
        pltpu.SemaphoreType.DMA,
    ],
)
def _sc_agg(pk_hbm, xw_hbm, out_hbm,
            cnt_v, elk_v, egx_v, kbuf0, kbuf1,
            rows0, rows1, agg_v, ksem0, ksem1, rsem0, rsem1):
    _sc_agg_body(pk_hbm, xw_hbm, out_hbm,
                 cnt_v, elk_v, egx_v, kbuf0, kbuf1,
                 rows0, rows1, agg_v, ksem0, ksem1, rsem0, rsem1)


def _epilogue_body(agg_ref, x_ref, root_ref, bias_ref, w1_ref, b1_ref,
                   w2_ref, b2_ref, w3_ref, b3_ref, out_ref):
    x = x_ref[:]
    ent = agg_ref[:] + jnp.dot(x, root_ref[:], preferred_element_type=jnp.float32)
    ent = ent + bias_ref[:] + x
    h = jnp.maximum(jnp.dot(ent, w1_ref[:], preferred_element_type=jnp.float32) + b1_ref[:], 0.0)
    p = jnp.dot(h, w2_ref[:], preferred_element_type=jnp.float32) + b2_ref[:]
    ent = p + ent
    out_ref[:] = jnp.dot(ent, w3_ref[:], preferred_element_type=jnp.float32) + b3_ref[:]


def _epilogue(agg, x, root, bias, W1, b1, W2, b2, W3, b3):
    full = lambda shape: pl.BlockSpec(shape, lambda i: (0,) * len(shape))
    return pl.pallas_call(
        _epilogue_body,
        grid=(N // _BLKN,),
        in_specs=[
            pl.BlockSpec((_BLKN, H), lambda i: (i, 0)),
            pl.BlockSpec((_BLKN, H), lambda i: (i, 0)),
            full((H, H)),
            full((1, H)),
            full((H, H // 2)),
            full((1, H // 2)),
            full((H // 2, H)),
            full((1, H)),
            full((H, HID)),
            full((1, HID)),
        ],
        out_specs=pl.BlockSpec((_BLKN, HID), lambda i: (i, 0)),
        out_shape=jax.ShapeDtypeStruct((N, HID), jnp.float32),
    )(agg, x, root, bias.reshape(1, H), W1, b1.reshape(1, H // 2),
      W2, b2.reshape(1, H), W3, b3.reshape(1, HID))


def kernel(node_embeds, basis, comp, root, rgcn_bias, W1, b1, W2, b2, W3, b3,
           edge_index, edge_type):
    x = node_embeds
    xw = _make_xw(comp, basis, x)
    pk = _make_keys(edge_index, edge_type)
    agg = _sc_agg(pk, xw)
    return _epilogue(agg, x, root, rgcn_bias, W1, b1, W2, b2, W3, b3)


# speedup vs baseline: 1.3331x; 1.3269x over previous
import functools

import jax
import jax.numpy as jnp
from jax import lax
from jax.experimental import pallas as pl
from jax.experimental.pallas import tpu as pltpu
from jax.experimental.pallas import tpu_sc as plsc

N = 10000
E = 320000
R = 16
NB = 4
H = 128
HID = 256

NC = 2
NS = 16
NT = NC * NS
NPT = 320
NPAD = NT * NPT
LK = NPT * R

CH = 4000
GCH = 128
ECAP = 128 * 94

_BLKN = 1000


def _xw_body(comp_ref, basis_ref, x_ref, out_ref):
    r = pl.program_id(0)
    w = comp_ref[r, 0] * basis_ref[0]
    for b in range(1, NB):
        w = w + comp_ref[r, b] * basis_ref[b]
    out_ref[:] = jnp.dot(x_ref[:], w, preferred_element_type=jnp.float32)


def _make_xw(comp, basis, x):
    nbn = N // _BLKN
    return pl.pallas_call(
        _xw_body,
        grid=(R, nbn),
        in_specs=[
            pl.BlockSpec(memory_space=pltpu.SMEM),
            pl.BlockSpec((NB, H, H), lambda r, n: (0, 0, 0)),
            pl.BlockSpec((_BLKN, H), lambda r, n: (n, 0)),
        ],
        out_specs=pl.BlockSpec((_BLKN, H), lambda r, n: (r * nbn + n, 0)),
        out_shape=jax.ShapeDtypeStruct((R * N, H), jnp.float32),
    )(comp, basis, x)


def _keys_body(ei_ref, et_ref, pk_ref):
    src = ei_ref[0:1, :]
    dst = ei_ref[1:2, :]
    et = et_ref[:]
    pk_ref[:] = (src << 18) | (dst * R + et)


def _make_keys(edge_index, edge_type):
    be = 12800
    pk = pl.pallas_call(
        _keys_body,
        grid=(E // be,),
        in_specs=[
            pl.BlockSpec((2, be), lambda i: (0, i)),
            pl.BlockSpec((1, be), lambda i: (0, i)),
        ],
        out_specs=pl.BlockSpec((1, be), lambda i: (0, i)),
        out_shape=jax.ShapeDtypeStruct((1, E), jnp.int32),
    )(edge_index, edge_type.reshape(1, E))
    return pk.reshape(E)


def _sc_agg_body(pk_hbm, xw_hbm, out_hbm,
                 cnt_v, elk_v, egx_v, kbuf0, kbuf1,
                 rows0, rows1, agg_v, ksem0, ksem1, rsem0, rsem1):
    wid = lax.axis_index("s") * NC + lax.axis_index("c")
    kbase = wid * LK
    base_node = wid * NPT
    kbufs = (kbuf0, kbuf1)
    rows = (rows0, rows1)
    ksems = (ksem0, ksem1)
    rsems = (rsem0, rsem1)

    zf = jnp.zeros((16,), jnp.float32)
    zi = jnp.zeros((16,), jnp.int32)
    lkfill = jnp.full((16,), LK, jnp.int32)

    def zero_cnt(i, _):
        cnt_v[pl.ds(i * 16, 16)] = zf
        return 0
    lax.fori_loop(0, (LK + 128) // 16, zero_cnt, 0)

    def zero_agg(i, _):
        agg_v[i // 8, pl.ds((i % 8) * 16, 16)] = zf
        return 0
    lax.fori_loop(0, (NPT + 1) * 8, zero_agg, 0)

    def init_lists(i, _):
        egx_v[pl.ds(i * 16, 16)] = zi
        elk_v[pl.ds(i * 16, 16)] = lkfill
        return 0
    lax.fori_loop(0, ECAP // 16, init_lists, 0)

    NCH1 = E // CH

    def p1_issue(c, par):
        pltpu.async_copy(pk_hbm.at[pl.ds(c * CH, CH)], kbufs[par],
                         ksems[par])

    p1_issue(0, 0)
    p1_issue(1, 1)

    def scan_pair(c2, nv):
        for par in range(2):
            c = 2 * c2 + par
            pltpu.make_async_copy(pk_hbm.at[pl.ds(c * CH, CH)],
                                  kbufs[par], ksems[par]).wait()
            kb = kbufs[par]

            @plsc.parallel_loop(0, CH // 16, 1, unroll=4, carry=nv)
            def nv(j, nv):
                p = kb[pl.ds(j * 16, 16)]
                k = p & 0x3FFFF
                g = (k & (R - 1)) * N + ((p >> 18) & 0x3FFF)
                lk = k - kbase
                m = (lk >= 0) & (lk < LK)
                plsc.store_compressed(elk_v.at[pl.ds(nv, 16)], lk, mask=m)
                plsc.store_compressed(egx_v.at[pl.ds(nv, 16)], g, mask=m)
                nm = plsc.all_reduce_population_count(m)
                return jnp.minimum(nv + nm[0], ECAP - 16)

            @pl.when(c + 2 < NCH1)
            def _():
                p1_issue(c + 2, par)
        return nv

    nv = lax.fori_loop(0, NCH1 // 2, scan_pair, jnp.int32(0))

    ng = (nv + 15) >> 4

    def cntloop(i, _):
        lkv = elk_v[pl.ds(i * 16, 16)]
        dupv, lastm = plsc.scan_count(lkv)
        plsc.addupdate_scatter(cnt_v, [lkv], dupv.astype(jnp.float32),
                               mask=lastm)
        return 0
    lax.fori_loop(0, ng, cntloop, 0)

    def invert(i, _):
        v = cnt_v[pl.ds(i * 16, 16)]
        cnt_v[pl.ds(i * 16, 16)] = 1.0 / jnp.maximum(v, 1.0)
        return 0
    lax.fori_loop(0, (LK + 128) // 16, invert, 0)

    nch = (nv + GCH - 1) // GCH

    def p2_issue(t, par):
        pltpu.async_copy(xw_hbm.at[egx_v.at[pl.ds(t * GCH, GCH)]],
                         rows[par], rsems[par])

    @pl.when(nch > 0)
    def _():
        p2_issue(0, 0)

    @pl.when(nch > 1)
    def _():
        p2_issue(1, 1)

    def proc_pair(t2, _):
        for par in range(2):
            t = 2 * t2 + par

            @pl.when(t < nch)
            def _():
                pltpu.make_async_copy(
                    xw_hbm.at[egx_v.at[pl.ds(t * GCH, GCH)]],
                    rows[par], rsems[par]).wait()
                off = t * GCH
                rv = rows[par]

                @plsc.parallel_loop(0, GCH // 16, 1, unroll=2)
                def grp(g):
                    lkv = elk_v[pl.ds(off + g * 16, 16)]
                    wv = plsc.load_gather(cnt_v, [lkv])
                    nodev = lkv >> 4
                    rbase = g * 16
                    for j in range(16):
                        node = nodev[j]
                        w = wv[j]
                        for h8 in range(8):
                            sl = pl.ds(h8 * 16, 16)
                            plsc.addupdate(agg_v.at[node, sl],
                                           w * rv[rbase + j, sl])

                @pl.when(t + 2 < nch)
                def _():
                    p2_issue(t + 2, par)
        return 0

    lax.fori_loop(0, (nch + 1) // 2, proc_pair, 0)

    pltpu.sync_copy(agg_v.at[pl.ds(0, NPT)], out_hbm.at[pl.ds(base_node, NPT)])


@functools.partial(
    pl.kernel,
    out_type=jax.ShapeDtypeStruct((NPAD, H), jnp.float32),
    mesh=plsc.VectorSubcoreMesh(core_axis_name="c", subcore_axis_name="s"),
    compiler_params=pltpu.CompilerParams(needs_layout_passes=False),
    scratch_types=[
        pltpu.VMEM((LK + 128,), jnp.float32),
        pltpu.VMEM((ECAP,), jnp.int32),
        pltpu.VMEM((ECAP,), jnp.int32),
        pltpu.VMEM((CH,), jnp.int32),
        pltpu.VMEM((CH,), jnp.int32),
        pltpu.VMEM((GCH, H), jnp.float32),
        pltpu.VMEM((GCH, H), jnp.float32),
        pltpu.VMEM((NPT + 1, H), jnp.float32),
        pltpu.SemaphoreType.DMA,
        pltpu.SemaphoreType.DMA,
        pltpu.SemaphoreType.DMA,
        pltpu.SemaphoreType.DMA,
    ],
)
def _sc_agg(pk_hbm, xw_hbm, out_hbm,
            cnt_v, elk_v, egx_v, kbuf0, kbuf1,
            rows0, rows1, agg_v, ksem0, ksem1, rsem0, rsem1):
    _sc_agg_body(pk_hbm, xw_hbm, out_hbm,
                 cnt_v, elk_v, egx_v, kbuf0, kbuf1,
                 rows0, rows1, agg_v, ksem0, ksem1, rsem0, rsem1)


def _epilogue_body(agg_ref, x_ref, root_ref, bias_ref, w1_ref, b1_ref,
                   w2_ref, b2_ref, w3_ref, b3_ref, out_ref):
    x = x_ref[:]
    ent = agg_ref[:] + jnp.dot(x, root_ref[:], preferred_element_type=jnp.float32)
    ent = ent + bias_ref[:] + x
    h = jnp.maximum(jnp.dot(ent, w1_ref[:], preferred_element_type=jnp.float32) + b1_ref[:], 0.0)
    p = jnp.dot(h, w2_ref[:], preferred_element_type=jnp.float32) + b2_ref[:]
    ent = p + ent
    out_ref[:] = jnp.dot(ent, w3_ref[:], preferred_element_type=jnp.float32) + b3_ref[:]


def _epilogue(agg, x, root, bias, W1, b1, W2, b2, W3, b3):
    full = lambda shape: pl.BlockSpec(shape, lambda i: (0,) * len(shape))
    return pl.pallas_call(
        _epilogue_body,
        grid=(N // _BLKN,),
        in_specs=[
            pl.BlockSpec((_BLKN, H), lambda i: (i, 0)),
            pl.BlockSpec((_BLKN, H), lambda i: (i, 0)),
            full((H, H)),
            full((1, H)),
            full((H, H // 2)),
            full((1, H // 2)),
            full((H // 2, H)),
            full((1, H)),
            full((H, HID)),
            full((1, HID)),
        ],
        out_specs=pl.BlockSpec((_BLKN, HID), lambda i: (i, 0)),
        out_shape=jax.ShapeDtypeStruct((N, HID), jnp.float32),
    )(agg, x, root, bias.reshape(1, H), W1, b1.reshape(1, H // 2),
      W2, b2.reshape(1, H), W3, b3.reshape(1, HID))


def kernel(node_embeds, basis, comp, root, rgcn_bias, W1, b1, W2, b2, W3, b3,
           edge_index, edge_type):
    x = node_embeds
    xw = _make_xw(comp, basis, x)
    pk = _make_keys(edge_index, edge_type)
    agg = _sc_agg(pk, xw)
    return _epilogue(agg, x, root, rgcn_bias, W1, b1, W2, b2, W3, b3)
